# baseline (speedup 1.0000x reference)
import functools

import jax
import jax.numpy as jnp
from jax import lax
from jax.experimental import pallas as pl
from jax.experimental.pallas import tpu as pltpu
from jax.experimental.pallas import tpu_sc as plsc

NUM_PUBLISHER = 1000000
EMBEDDING_DIM = 32
BATCH = 16384

_NC = 2
_NS = 16
_NW = _NC * _NS
_B_PER_W = BATCH // _NW
_CHUNK = 128
_NCHUNK = _B_PER_W // _CHUNK


def _gather_body(table_hbm, idx_hbm, out_hbm, idx_v, rows_v, sem):
    wid = lax.axis_index("s") * _NC + lax.axis_index("c")
    base = wid * _B_PER_W
    pltpu.sync_copy(idx_hbm.at[wid], idx_v)
    copies = [
        pltpu.async_copy(
            table_hbm.at[idx_v.at[j]],
            rows_v.at[pl.ds(j * _CHUNK, _CHUNK)],
            sem,
        )
        for j in range(_NCHUNK)
    ]
    for c in copies:
        c.wait()
    pltpu.sync_copy(rows_v, out_hbm.at[pl.ds(base, _B_PER_W)])


@jax.jit
def _embedding_lookup(table, idx3):
    mesh = plsc.VectorSubcoreMesh(core_axis_name="c", subcore_axis_name="s")
    run = functools.partial(
        pl.kernel,
        out_type=jax.ShapeDtypeStruct((BATCH, EMBEDDING_DIM), jnp.float32),
        mesh=mesh,
        scratch_types=[
            pltpu.VMEM((_NCHUNK, _CHUNK), jnp.int32),
            pltpu.VMEM((_B_PER_W, EMBEDDING_DIM), jnp.float32),
            pltpu.SemaphoreType.DMA,
        ],
        compiler_params=pltpu.CompilerParams(use_tc_tiling_on_sc=False),
    )(_gather_body)
    return run(table, idx3)


def kernel(item_fea, embedding_publisher):
    idx3 = item_fea[:, 0].reshape(_NW, _NCHUNK, _CHUNK)
    return _embedding_lookup(embedding_publisher, idx3)


# speedup vs baseline: 7.5505x; 7.5505x over previous
import functools

import jax
import jax.numpy as jnp
from jax import lax
from jax.experimental import pallas as pl
from jax.experimental.pallas import tpu as pltpu
from jax.experimental.pallas import tpu_sc as plsc

NUM_PUBLISHER = 1000000
EMBEDDING_DIM = 32
BATCH = 16384

_NC = 2
_NS = 16
_NW = _NC * _NS
_B_PER_W = BATCH // _NW
_LANES = 16
_NWAVE = _B_PER_W // _LANES
_BLK = 16


def _gather_body(tab_hbm, idx_hbm, out_hbm, idx_v, buf_v, comp_v, sem):
    w = lax.axis_index("s") * _NC + lax.axis_index("c")
    base = w * _B_PER_W
    pltpu.sync_copy(idx_hbm.at[w], idx_v)
    lane = lax.iota(jnp.int32, _LANES)
    halfsel = lane // 8
    colbase = 16 * (lane % 8)

    def wave(g, carry):
        vec = idx_v[pl.ds(g * _LANES, _LANES)]
        p128v = vec & jnp.int32(~127)
        sv = (vec >> 4) & jnp.int32(7)
        rem = vec & jnp.int32(15)
        for k in range(_LANES):
            tile_ref = tab_hbm.at[
                :, :, pl.ds(pl.multiple_of(p128v[k], 128), 128)
            ]
            pltpu.async_copy(
                tile_ref.at[:, :, pl.ds(pl.multiple_of(sv[k] * 16, 16), _BLK)],
                buf_v.at[k // 8, :, :, pl.ds(16 * (k % 8), _BLK)],
                sem,
            )
        for _ in range(_LANES):
            pltpu.make_async_copy(
                tab_hbm.at[:, :, pl.ds(0, _BLK)],
                buf_v.at[0, :, :, pl.ds(0, _BLK)],
                sem,
            ).wait()
        colv = colbase + rem
        q = g // 8
        off = 16 * (g % 8)
        for c in range(EMBEDDING_DIM):
            ctv = jnp.full((_LANES,), c // 8, jnp.int32)
            csv = jnp.full((_LANES,), c % 8, jnp.int32)
            gathered = plsc.load_gather(buf_v, [halfsel, ctv, csv, colv])
            comp_v[q, c, pl.ds(off, _LANES)] = gathered
        return carry

    lax.fori_loop(0, _NWAVE, wave, 0)
    for q in range(4):
        pltpu.sync_copy(
            comp_v.at[q], out_hbm.at[:, pl.ds(base + 128 * q, 128)]
        )


@jax.jit
def _embedding_lookup(tab3, idx2):
    mesh = plsc.VectorSubcoreMesh(core_axis_name="c", subcore_axis_name="s")
    run = functools.partial(
        pl.kernel,
        out_type=jax.ShapeDtypeStruct((EMBEDDING_DIM, BATCH), jnp.float32),
        mesh=mesh,
        scratch_types=[
            pltpu.VMEM((_B_PER_W,), jnp.int32),
            pltpu.VMEM((2, 4, 8, 128), jnp.float32),
            pltpu.VMEM((4, EMBEDDING_DIM, 128), jnp.float32),
            pltpu.SemaphoreType.DMA,
        ],
        compiler_params=pltpu.CompilerParams(
            use_tc_tiling_on_sc=True, needs_layout_passes=False
        ),
    )(_gather_body)
    return run(tab3, idx2)


def kernel(item_fea, embedding_publisher):
    tab3 = embedding_publisher.T.reshape(4, 8, NUM_PUBLISHER)
    idx2 = item_fea[:, 0].reshape(_NW, _B_PER_W)
    out_t = _embedding_lookup(tab3, idx2)
    return out_t.T


# speedup vs baseline: 9.0991x; 1.2051x over previous
import functools

import jax
import jax.numpy as jnp
from jax import lax
from jax.experimental import pallas as pl
from jax.experimental.pallas import tpu as pltpu
from jax.experimental.pallas import tpu_sc as plsc

NUM_PUBLISHER = 1000000
EMBEDDING_DIM = 32
BATCH = 16384

_NC = 2
_NS = 16
_NW = _NC * _NS
_B_PER_W = BATCH // _NW
_LANES = 16
_NWAVE = _B_PER_W // _LANES
_BLK = 16
_NBUF = 4


def _gather_body(tab_hbm, idx_hbm, out_hbm, idx_v, buf_v, comp_v, sems):
    w = lax.axis_index("s") * _NC + lax.axis_index("c")
    base = w * _B_PER_W
    pltpu.sync_copy(idx_hbm.at[w], idx_v)
    lane = lax.iota(jnp.int32, _LANES)
    halfsel = lane // 8
    colbase = 16 * (lane % 8)

    def fire(gw, slot):
        vec = idx_v[pl.ds(gw * _LANES, _LANES)]
        p128v = vec & jnp.int32(~127)
        sv = (vec >> 4) & jnp.int32(7)
        for k in range(_LANES):
            tile_ref = tab_hbm.at[
                :, :, pl.ds(pl.multiple_of(p128v[k], 128), 128)
            ]
            pltpu.async_copy(
                tile_ref.at[:, :, pl.ds(pl.multiple_of(sv[k] * 16, 16), _BLK)],
                buf_v.at[slot, k // 8, :, :, pl.ds(16 * (k % 8), _BLK)],
                sems.at[slot],
            )

    def compact(gw, slot):
        for _ in range(_LANES):
            pltpu.make_async_copy(
                tab_hbm.at[:, :, pl.ds(0, _BLK)],
                buf_v.at[slot, 0, :, :, pl.ds(0, _BLK)],
                sems.at[slot],
            ).wait()
        vec = idx_v[pl.ds(gw * _LANES, _LANES)]
        colv = colbase + (vec & jnp.int32(15))
        slotv = jnp.full((_LANES,), 1, jnp.int32) * slot
        q = gw // 8
        off = 16 * (gw % 8)
        for c in range(EMBEDDING_DIM):
            ctv = jnp.full((_LANES,), c // 8, jnp.int32)
            csv = jnp.full((_LANES,), c % 8, jnp.int32)
            gathered = plsc.load_gather(
                buf_v, [slotv, halfsel, ctv, csv, colv]
            )
            comp_v[q, c, pl.ds(off, _LANES)] = gathered

    for s in range(_NBUF - 1):
        fire(s, s)

    def outer(g, carry):
        slot = g % _NBUF
        gf = g + _NBUF - 1

        @pl.when(gf < _NWAVE)
        def _():
            fire(gf, gf % _NBUF)

        compact(g, slot)
        return carry

    lax.fori_loop(0, _NWAVE, outer, 0)
    for q in range(4):
        pltpu.sync_copy(
            comp_v.at[q], out_hbm.at[:, pl.ds(base + 128 * q, 128)]
        )


@jax.jit
def _embedding_lookup(tab3, idx2):
    mesh = plsc.VectorSubcoreMesh(core_axis_name="c", subcore_axis_name="s")
    run = functools.partial(
        pl.kernel,
        out_type=jax.ShapeDtypeStruct((EMBEDDING_DIM, BATCH), jnp.float32),
        mesh=mesh,
        scratch_types=[
            pltpu.VMEM((_B_PER_W,), jnp.int32),
            pltpu.VMEM((_NBUF, 2, 4, 8, 128), jnp.float32),
            pltpu.VMEM((4, EMBEDDING_DIM, 128), jnp.float32),
            pltpu.SemaphoreType.DMA((_NBUF,)),
        ],
        compiler_params=pltpu.CompilerParams(
            use_tc_tiling_on_sc=True, needs_layout_passes=False
        ),
    )(_gather_body)
    return run(tab3, idx2)


def kernel(item_fea, embedding_publisher):
    tab3 = embedding_publisher.T.reshape(4, 8, NUM_PUBLISHER)
    idx2 = item_fea[:, 0].reshape(_NW, _B_PER_W)
    out_t = _embedding_lookup(tab3, idx2)
    return out_t.T


# speedup vs baseline: 10.4823x; 1.1520x over previous
import functools

import jax
import jax.numpy as jnp
from jax import lax
from jax.experimental import pallas as pl
from jax.experimental.pallas import tpu as pltpu
from jax.experimental.pallas import tpu_sc as plsc

NUM_PUBLISHER = 1000000
EMBEDDING_DIM = 32
BATCH = 16384

_NC = 2
_NS = 16
_NW = _NC * _NS
_B_PER_W = BATCH // _NW
_LANES = 16
_NWAVE = _B_PER_W // _LANES
_BLK = 16
_NBUF = 8


def _gather_body(tab_hbm, idx_hbm, out_hbm, idx_v, buf_v, comp_v, sems):
    w = lax.axis_index("s") * _NC + lax.axis_index("c")
    base = w * _B_PER_W
    pltpu.sync_copy(idx_hbm.at[w], idx_v)
    lane = lax.iota(jnp.int32, _LANES)
    halfsel = lane // 8
    colbase = 16 * (lane % 8)

    def fire(gw, slot):
        vec = idx_v[pl.ds(gw * _LANES, _LANES)]
        p128v = vec & jnp.int32(~127)
        sv = (vec >> 4) & jnp.int32(7)
        for k in range(_LANES):
            tile_ref = tab_hbm.at[
                :, :, pl.ds(pl.multiple_of(p128v[k], 128), 128)
            ]
            pltpu.async_copy(
                tile_ref.at[:, :, pl.ds(pl.multiple_of(sv[k] * 16, 16), _BLK)],
                buf_v.at[slot, k // 8, :, :, pl.ds(16 * (k % 8), _BLK)],
                sems.at[slot],
            )

    def compact(gw, slot):
        for h in range(2):
            pltpu.make_async_copy(
                tab_hbm.at[:, :, pl.ds(0, 128)],
                buf_v.at[slot, h],
                sems.at[slot],
            ).wait()
        vec = idx_v[pl.ds(gw * _LANES, _LANES)]
        colv = colbase + (vec & jnp.int32(15))
        slotv = jnp.full((_LANES,), 1, jnp.int32) * slot
        q = gw // 8
        off = 16 * (gw % 8)
        for c in range(EMBEDDING_DIM):
            ctv = jnp.full((_LANES,), c // 8, jnp.int32)
            csv = jnp.full((_LANES,), c % 8, jnp.int32)
            gathered = plsc.load_gather(
                buf_v, [slotv, halfsel, ctv, csv, colv]
            )
            comp_v[q, c, pl.ds(off, _LANES)] = gathered

    def prologue(s, carry):
        fire(s, s)
        return carry

    lax.fori_loop(0, _NBUF - 1, prologue, 0)

    def outer(g, carry):
        slot = g % _NBUF
        gf = g + _NBUF - 1

        @pl.when(gf < _NWAVE)
        def _():
            fire(gf, gf % _NBUF)

        compact(g, slot)
        return carry

    lax.fori_loop(0, _NWAVE, outer, 0)
    for q in range(4):
        pltpu.sync_copy(
            comp_v.at[q], out_hbm.at[:, pl.ds(base + 128 * q, 128)]
        )


@jax.jit
def _embedding_lookup(tab3, idx2):
    mesh = plsc.VectorSubcoreMesh(core_axis_name="c", subcore_axis_name="s")
    run = functools.partial(
        pl.kernel,
        out_type=jax.ShapeDtypeStruct((EMBEDDING_DIM, BATCH), jnp.float32),
        mesh=mesh,
        scratch_types=[
            pltpu.VMEM((_B_PER_W,), jnp.int32),
            pltpu.VMEM((_NBUF, 2, 4, 8, 128), jnp.float32),
            pltpu.VMEM((4, EMBEDDING_DIM, 128), jnp.float32),
            pltpu.SemaphoreType.DMA((_NBUF,)),
        ],
        compiler_params=pltpu.CompilerParams(
            use_tc_tiling_on_sc=True, needs_layout_passes=False
        ),
    )(_gather_body)
    return run(tab3, idx2)


def kernel(item_fea, embedding_publisher):
    tab3 = embedding_publisher.T.reshape(4, 8, NUM_PUBLISHER)
    idx2 = item_fea[:, 0].reshape(_NW, _B_PER_W)
    out_t = _embedding_lookup(tab3, idx2)
    return out_t.T
